# baseline (speedup 1.0000x reference)
import functools

import jax
import jax.numpy as jnp
from jax import lax
from jax.experimental import pallas as pl
from jax.experimental.pallas import tpu as pltpu
from jax.experimental.pallas import tpu_sc as plsc

MC = 5
B = 32
Q = 8192
C = 16
CONF_THR = 0.15

BB = 8
QB = 4096

_NEG_INF = float("-inf")


def _dense_body(z_ref, labels_ref, valid_ref,
                ml_ref, unc_ref, mconf_ref, munc_ref):
    z = z_ref[...]
    e = jnp.exp(z)
    s = jnp.sum(e, axis=2)

    macc = jnp.zeros((BB, C, QB), jnp.float32)
    sacc = jnp.zeros((BB, C, QB), jnp.float32)
    for m in range(MC):
        t = e[m] / s[m][:, None, :]
        macc = macc + t
        sacc = sacc + t * t
    mean_p = macc * jnp.float32(1.0 / MC)
    w = sacc - mean_p * macc

    unc = jnp.sum(w, axis=1) * jnp.float32(1.0 / (C * (MC - 1)))
    conf = jnp.max(mean_p, axis=1)
    oh = mean_p == conf[:, None, :]
    cidx = lax.broadcasted_iota(jnp.int32, (BB, C, QB), 1)
    pred = jnp.min(jnp.where(oh, cidx, jnp.int32(C)), axis=1)

    lab = labels_ref[...]
    vm = valid_ref[...] != 0
    hcc = (pred == lab) & vm & (conf > CONF_THR)

    ml_ref[...] = ((z[0] + z[1] + z[2] + z[3] + z[4])
                   * jnp.float32(1.0 / MC))
    unc_ref[...] = unc
    mconf_ref[...] = jnp.where(hcc, conf, _NEG_INF)
    munc_ref[...] = jnp.where(vm, unc, _NEG_INF)


def _dense_call(zt, labels32, vmask32):
    grid = (B // BB, Q // QB)
    bq_spec = pl.BlockSpec((BB, QB), lambda i, j: (i, j))
    bq_shape = jax.ShapeDtypeStruct((B, Q), jnp.float32)
    return pl.pallas_call(
        _dense_body,
        grid=grid,
        in_specs=[
            pl.BlockSpec((MC, BB, C, QB), lambda i, j: (0, i, 0, j)),
            bq_spec,
            bq_spec,
        ],
        out_specs=[
            pl.BlockSpec((BB, C, QB), lambda i, j: (i, 0, j)),
            bq_spec,
            bq_spec,
            bq_spec,
        ],
        out_shape=[
            jax.ShapeDtypeStruct((B, C, Q), jnp.float32),
            bq_shape,
            bq_shape,
            bq_shape,
        ],
        compiler_params=pltpu.CompilerParams(
            dimension_semantics=("parallel", "parallel"),
        ),
    )(zt, labels32, vmask32)


def _lane_argmax(best_v, best_i):
    m = jnp.max(best_v, axis=0)
    cand = jnp.where(best_v == m, best_i, jnp.int32(2**30))
    return m, jnp.min(cand, axis=0)


def _select_body(mconf_hbm, munc_hbm, out_hbm, conf_v, unc_v, out_v):
    wid = lax.axis_index("s") * 2 + lax.axis_index("c")
    pltpu.sync_copy(mconf_hbm.at[wid], conf_v)
    pltpu.sync_copy(munc_hbm.at[wid], unc_v)

    iota = lax.iota(jnp.int32, 16)
    nchunk = Q // 16
    big = jnp.full((16,), jnp.int32(2**30))
    neg = jnp.full((16,), _NEG_INF, jnp.float32)

    def upd(v, q, bv, bi):
        take = (v > bv) | ((v == bv) & (q < bi))
        return jnp.where(take, v, bv), jnp.where(take, q, bi)

    def pass1(i, carry):
        bc_v, bc_i, bu_v, bu_i = carry
        base = i * 16
        q = iota + base
        cv = conf_v[pl.ds(base, 16)]
        uv = unc_v[pl.ds(base, 16)]
        bc_v, bc_i = upd(cv, q, bc_v, bc_i)
        bu_v, bu_i = upd(uv, q, bu_v, bu_i)
        return bc_v, bc_i, bu_v, bu_i

    bc_v, bc_i, bu_v, bu_i = lax.fori_loop(
        0, nchunk, pass1, (neg, big, neg, big))

    conf_max, pos0 = _lane_argmax(bc_v, bc_i)
    _, neg0 = _lane_argmax(bu_v, bu_i)

    def pass2(i, carry):
        bu_v, bu_i = carry
        base = i * 16
        q = iota + base
        uv = unc_v[pl.ds(base, 16)]
        uv = jnp.where(q == neg0, _NEG_INF, uv)
        return upd(uv, q, bu_v, bu_i)

    bu_v, bu_i = lax.fori_loop(0, nchunk, pass2, (neg, big))
    _, neg1 = _lane_argmax(bu_v, bu_i)

    has_pos = conf_max > jnp.float32(0.0)
    pos = jnp.where(has_pos, pos0, jnp.int32(-1))

    r = jnp.where(iota == 0, pos,
        jnp.where(iota == 1, has_pos.astype(jnp.int32),
        jnp.where(iota == 2, neg0,
        jnp.where(iota == 3, neg1, jnp.int32(0)))))
    out_v[...] = r
    pltpu.sync_copy(out_v, out_hbm.at[wid])


@functools.cache
def _select_call():
    return functools.partial(
        pl.kernel,
        out_type=jax.ShapeDtypeStruct((B, 16), jnp.int32),
        mesh=plsc.VectorSubcoreMesh(core_axis_name="c", subcore_axis_name="s"),
        compiler_params=pltpu.CompilerParams(needs_layout_passes=False),
        scratch_types=[
            pltpu.VMEM((Q,), jnp.float32),
            pltpu.VMEM((Q,), jnp.float32),
            pltpu.VMEM((16,), jnp.int32),
        ],
    )(_select_body)


def kernel(all_logits, labels, valid_mask):
    zt = jnp.transpose(all_logits, (0, 1, 3, 2))
    labels32 = labels.astype(jnp.int32)
    vmask32 = valid_mask.astype(jnp.int32)
    ml, unc, mconf, munc = _dense_call(zt, labels32, vmask32)
    sel = _select_call()(mconf, munc)
    pos_idx = sel[:, 0]
    has_pos = sel[:, 1].astype(jnp.bool_)
    neg_idx = sel[:, 2:4]
    mean_logits = jnp.transpose(ml, (0, 2, 1))
    return (mean_logits, unc, pos_idx, has_pos, neg_idx)


# speedup vs baseline: 1.0011x; 1.0011x over previous
import functools

import jax
import jax.numpy as jnp
from jax import lax
from jax.experimental import pallas as pl
from jax.experimental.pallas import tpu as pltpu
from jax.experimental.pallas import tpu_sc as plsc

MC = 5
B = 32
Q = 8192
C = 16
CONF_THR = 0.15

BB = 16
QB = 2048

_NEG_INF = float("-inf")


def _dense_body(z_ref, labels_ref, valid_ref,
                ml_ref, unc_ref, mconf_ref, munc_ref):
    z = z_ref[...]
    e = jnp.exp(z)
    s = jnp.sum(e, axis=2)

    macc = jnp.zeros((BB, C, QB), jnp.float32)
    sacc = jnp.zeros((BB, C, QB), jnp.float32)
    for m in range(MC):
        t = e[m] / s[m][:, None, :]
        macc = macc + t
        sacc = sacc + t * t
    mean_p = macc * jnp.float32(1.0 / MC)
    w = sacc - mean_p * macc

    unc = jnp.sum(w, axis=1) * jnp.float32(1.0 / (C * (MC - 1)))
    conf = jnp.max(mean_p, axis=1)
    oh = mean_p == conf[:, None, :]
    cidx = lax.broadcasted_iota(jnp.int32, (BB, C, QB), 1)
    pred = jnp.min(jnp.where(oh, cidx, jnp.int32(C)), axis=1)

    lab = labels_ref[...]
    vm = valid_ref[...] != 0
    hcc = (pred == lab) & vm & (conf > CONF_THR)

    ml_ref[...] = ((z[0] + z[1] + z[2] + z[3] + z[4])
                   * jnp.float32(1.0 / MC))
    unc_ref[...] = unc
    mconf_ref[...] = jnp.where(hcc, conf, _NEG_INF)
    munc_ref[...] = jnp.where(vm, unc, _NEG_INF)


def _dense_call(zt, labels32, vmask32):
    grid = (B // BB, Q // QB)
    bq_spec = pl.BlockSpec((BB, QB), lambda i, j: (i, j))
    bq_shape = jax.ShapeDtypeStruct((B, Q), jnp.float32)
    return pl.pallas_call(
        _dense_body,
        grid=grid,
        in_specs=[
            pl.BlockSpec((MC, BB, C, QB), lambda i, j: (0, i, 0, j)),
            bq_spec,
            bq_spec,
        ],
        out_specs=[
            pl.BlockSpec((BB, C, QB), lambda i, j: (i, 0, j)),
            bq_spec,
            bq_spec,
            bq_spec,
        ],
        out_shape=[
            jax.ShapeDtypeStruct((B, C, Q), jnp.float32),
            bq_shape,
            bq_shape,
            bq_shape,
        ],
        compiler_params=pltpu.CompilerParams(
            dimension_semantics=("parallel", "parallel"),
        ),
    )(zt, labels32, vmask32)


def _lane_argmax(best_v, best_i):
    m = jnp.max(best_v, axis=0)
    cand = jnp.where(best_v == m, best_i, jnp.int32(2**30))
    return m, jnp.min(cand, axis=0)


def _select_body(mconf_hbm, munc_hbm, out_hbm, conf_v, unc_v, out_v):
    wid = lax.axis_index("s") * 2 + lax.axis_index("c")
    pltpu.sync_copy(mconf_hbm.at[wid], conf_v)
    pltpu.sync_copy(munc_hbm.at[wid], unc_v)

    iota = lax.iota(jnp.int32, 16)
    nchunk = Q // 16
    big = jnp.full((16,), jnp.int32(2**30))
    neg = jnp.full((16,), _NEG_INF, jnp.float32)

    def upd(v, q, bv, bi):
        take = (v > bv) | ((v == bv) & (q < bi))
        return jnp.where(take, v, bv), jnp.where(take, q, bi)

    def pass1(i, carry):
        bc_v, bc_i, bu_v, bu_i = carry
        base = i * 16
        q = iota + base
        cv = conf_v[pl.ds(base, 16)]
        uv = unc_v[pl.ds(base, 16)]
        bc_v, bc_i = upd(cv, q, bc_v, bc_i)
        bu_v, bu_i = upd(uv, q, bu_v, bu_i)
        return bc_v, bc_i, bu_v, bu_i

    bc_v, bc_i, bu_v, bu_i = lax.fori_loop(
        0, nchunk, pass1, (neg, big, neg, big))

    conf_max, pos0 = _lane_argmax(bc_v, bc_i)
    _, neg0 = _lane_argmax(bu_v, bu_i)

    def pass2(i, carry):
        bu_v, bu_i = carry
        base = i * 16
        q = iota + base
        uv = unc_v[pl.ds(base, 16)]
        uv = jnp.where(q == neg0, _NEG_INF, uv)
        return upd(uv, q, bu_v, bu_i)

    bu_v, bu_i = lax.fori_loop(0, nchunk, pass2, (neg, big))
    _, neg1 = _lane_argmax(bu_v, bu_i)

    has_pos = conf_max > jnp.float32(0.0)
    pos = jnp.where(has_pos, pos0, jnp.int32(-1))

    r = jnp.where(iota == 0, pos,
        jnp.where(iota == 1, has_pos.astype(jnp.int32),
        jnp.where(iota == 2, neg0,
        jnp.where(iota == 3, neg1, jnp.int32(0)))))
    out_v[...] = r
    pltpu.sync_copy(out_v, out_hbm.at[wid])


@functools.cache
def _select_call():
    return functools.partial(
        pl.kernel,
        out_type=jax.ShapeDtypeStruct((B, 16), jnp.int32),
        mesh=plsc.VectorSubcoreMesh(core_axis_name="c", subcore_axis_name="s"),
        compiler_params=pltpu.CompilerParams(needs_layout_passes=False),
        scratch_types=[
            pltpu.VMEM((Q,), jnp.float32),
            pltpu.VMEM((Q,), jnp.float32),
            pltpu.VMEM((16,), jnp.int32),
        ],
    )(_select_body)


def kernel(all_logits, labels, valid_mask):
    zt = jnp.transpose(all_logits, (0, 1, 3, 2))
    labels32 = labels.astype(jnp.int32)
    vmask32 = valid_mask.astype(jnp.int32)
    ml, unc, mconf, munc = _dense_call(zt, labels32, vmask32)
    sel = _select_call()(mconf, munc)
    pos_idx = sel[:, 0]
    has_pos = sel[:, 1].astype(jnp.bool_)
    neg_idx = sel[:, 2:4]
    mean_logits = jnp.transpose(ml, (0, 2, 1))
    return (mean_logits, unc, pos_idx, has_pos, neg_idx)


# speedup vs baseline: 1.0254x; 1.0243x over previous
import functools

import jax
import jax.numpy as jnp
from jax import lax
from jax.experimental import pallas as pl
from jax.experimental.pallas import tpu as pltpu
from jax.experimental.pallas import tpu_sc as plsc

MC = 5
B = 32
Q = 8192
C = 16
CONF_THR = 0.15

BB = 8
QB = 2048

_NEG_INF = float("-inf")


def _dense_body(z_ref, labels_ref, valid_ref,
                ml_ref, unc_ref, mconf_ref, munc_ref):
    z = z_ref[...]
    e = jnp.exp(z)
    s = jnp.sum(e, axis=2)

    macc = jnp.zeros((BB, C, QB), jnp.float32)
    sacc = jnp.zeros((BB, C, QB), jnp.float32)
    for m in range(MC):
        t = e[m] / s[m][:, None, :]
        macc = macc + t
        sacc = sacc + t * t
    mean_p = macc * jnp.float32(1.0 / MC)
    w = sacc - mean_p * macc

    unc = jnp.sum(w, axis=1) * jnp.float32(1.0 / (C * (MC - 1)))
    conf = jnp.max(mean_p, axis=1)
    oh = mean_p == conf[:, None, :]
    cidx = lax.broadcasted_iota(jnp.int32, (BB, C, QB), 1)
    pred = jnp.min(jnp.where(oh, cidx, jnp.int32(C)), axis=1)

    lab = labels_ref[...]
    vm = valid_ref[...] != 0
    hcc = (pred == lab) & vm & (conf > CONF_THR)

    ml_ref[...] = ((z[0] + z[1] + z[2] + z[3] + z[4])
                   * jnp.float32(1.0 / MC))
    unc_ref[...] = unc
    mconf_ref[...] = jnp.where(hcc, conf, _NEG_INF)
    munc_ref[...] = jnp.where(vm, unc, _NEG_INF)


def _dense_call(zt, labels32, vmask32):
    grid = (B // BB, Q // QB)
    bq_spec = pl.BlockSpec((BB, QB), lambda i, j: (i, j))
    bq_shape = jax.ShapeDtypeStruct((B, Q), jnp.float32)
    return pl.pallas_call(
        _dense_body,
        grid=grid,
        in_specs=[
            pl.BlockSpec((MC, BB, C, QB), lambda i, j: (0, i, 0, j)),
            bq_spec,
            bq_spec,
        ],
        out_specs=[
            pl.BlockSpec((BB, C, QB), lambda i, j: (i, 0, j)),
            bq_spec,
            bq_spec,
            bq_spec,
        ],
        out_shape=[
            jax.ShapeDtypeStruct((B, C, Q), jnp.float32),
            bq_shape,
            bq_shape,
            bq_shape,
        ],
        compiler_params=pltpu.CompilerParams(
            dimension_semantics=("parallel", "parallel"),
        ),
    )(zt, labels32, vmask32)


def _lane_argmax(best_v, best_i):
    m = jnp.max(best_v, axis=0)
    cand = jnp.where(best_v == m, best_i, jnp.int32(2**30))
    return m, jnp.min(cand, axis=0)


def _select_body(mconf_hbm, munc_hbm, out_hbm, conf_v, unc_v, out_v):
    wid = lax.axis_index("s") * 2 + lax.axis_index("c")
    pltpu.sync_copy(mconf_hbm.at[wid], conf_v)
    pltpu.sync_copy(munc_hbm.at[wid], unc_v)

    iota = lax.iota(jnp.int32, 16)
    nchunk = Q // 16
    big = jnp.full((16,), jnp.int32(2**30))
    neg = jnp.full((16,), _NEG_INF, jnp.float32)

    def upd(v, q, bv, bi):
        take = (v > bv) | ((v == bv) & (q < bi))
        return jnp.where(take, v, bv), jnp.where(take, q, bi)

    def pass1(i, carry):
        bc_v, bc_i, bu_v, bu_i = carry
        base = i * 16
        q = iota + base
        cv = conf_v[pl.ds(base, 16)]
        uv = unc_v[pl.ds(base, 16)]
        bc_v, bc_i = upd(cv, q, bc_v, bc_i)
        bu_v, bu_i = upd(uv, q, bu_v, bu_i)
        return bc_v, bc_i, bu_v, bu_i

    bc_v, bc_i, bu_v, bu_i = lax.fori_loop(
        0, nchunk, pass1, (neg, big, neg, big), unroll=8)

    conf_max, pos0 = _lane_argmax(bc_v, bc_i)
    _, neg0 = _lane_argmax(bu_v, bu_i)

    def pass2(i, carry):
        bu_v, bu_i = carry
        base = i * 16
        q = iota + base
        uv = unc_v[pl.ds(base, 16)]
        uv = jnp.where(q == neg0, _NEG_INF, uv)
        return upd(uv, q, bu_v, bu_i)

    bu_v, bu_i = lax.fori_loop(0, nchunk, pass2, (neg, big), unroll=8)
    _, neg1 = _lane_argmax(bu_v, bu_i)

    has_pos = conf_max > jnp.float32(0.0)
    pos = jnp.where(has_pos, pos0, jnp.int32(-1))

    r = jnp.where(iota == 0, pos,
        jnp.where(iota == 1, has_pos.astype(jnp.int32),
        jnp.where(iota == 2, neg0,
        jnp.where(iota == 3, neg1, jnp.int32(0)))))
    out_v[...] = r
    pltpu.sync_copy(out_v, out_hbm.at[wid])


@functools.cache
def _select_call():
    return functools.partial(
        pl.kernel,
        out_type=jax.ShapeDtypeStruct((B, 16), jnp.int32),
        mesh=plsc.VectorSubcoreMesh(core_axis_name="c", subcore_axis_name="s"),
        compiler_params=pltpu.CompilerParams(needs_layout_passes=False),
        scratch_types=[
            pltpu.VMEM((Q,), jnp.float32),
            pltpu.VMEM((Q,), jnp.float32),
            pltpu.VMEM((16,), jnp.int32),
        ],
    )(_select_body)


def kernel(all_logits, labels, valid_mask):
    zt = jnp.transpose(all_logits, (0, 1, 3, 2))
    labels32 = labels.astype(jnp.int32)
    vmask32 = valid_mask.astype(jnp.int32)
    ml, unc, mconf, munc = _dense_call(zt, labels32, vmask32)
    sel = _select_call()(mconf, munc)
    pos_idx = sel[:, 0]
    has_pos = sel[:, 1].astype(jnp.bool_)
    neg_idx = sel[:, 2:4]
    mean_logits = jnp.transpose(ml, (0, 2, 1))
    return (mean_logits, unc, pos_idx, has_pos, neg_idx)
